# baseline (speedup 1.0000x reference)
import functools

import jax
import jax.numpy as jnp
from jax import lax
from jax.experimental import pallas as pl
from jax.experimental.pallas import tpu as pltpu
from jax.experimental.pallas import tpu_sc as plsc

EXPERT_DIM = 64
BATCH = 16384

NC = 2
NS = 16
NW = NC * NS
CHUNK = 128
ROWS_PER_W = BATCH // NW
N_CHUNK = ROWS_PER_W // CHUNK


def _gather_body(table_hbm, idx_hbm, out_hbm, idx_v, rows_v, sem):
    wid = lax.axis_index("s") * NC + lax.axis_index("c")
    blk = wid * N_CHUNK
    pltpu.sync_copy(idx_hbm.at[pl.ds(blk, N_CHUNK)], idx_v)
    copies = [
        pltpu.async_copy(
            table_hbm.at[idx_v.at[j]],
            rows_v.at[pl.ds(j * CHUNK, CHUNK)],
            sem,
        )
        for j in range(N_CHUNK)
    ]
    for c in copies:
        c.wait()
    pltpu.sync_copy(rows_v, out_hbm.at[pl.ds(wid * ROWS_PER_W, ROWS_PER_W)])


@functools.cache
def _sc_gather_fn():
    return pl.kernel(
        _gather_body,
        out_type=jax.ShapeDtypeStruct((BATCH, EXPERT_DIM), jnp.float32),
        mesh=plsc.VectorSubcoreMesh(
            core_axis_name="c", subcore_axis_name="s", num_cores=NC, num_subcores=NS
        ),
        scratch_types=[
            pltpu.VMEM((N_CHUNK, CHUNK), jnp.int32),
            pltpu.VMEM((ROWS_PER_W, EXPERT_DIM), jnp.float32),
            pltpu.SemaphoreType.DMA,
        ],
        compiler_params=pltpu.CompilerParams(use_tc_tiling_on_sc=False),
    )


def _linear_body(x_ref, w_ref, b_ref, o_ref):
    o_ref[...] = (
        lax.dot_general(
            x_ref[...],
            w_ref[...],
            (((1,), (1,)), ((), ())),
            preferred_element_type=jnp.float32,
        )
        + b_ref[...]
    )


_BLK = 2048


def _tc_linear(x, W, b2d):
    return pl.pallas_call(
        _linear_body,
        grid=(BATCH // _BLK,),
        in_specs=[
            pl.BlockSpec((_BLK, EXPERT_DIM), lambda i: (i, 0)),
            pl.BlockSpec((EXPERT_DIM, EXPERT_DIM), lambda i: (0, 0)),
            pl.BlockSpec((1, EXPERT_DIM), lambda i: (0, 0)),
        ],
        out_specs=pl.BlockSpec((_BLK, EXPERT_DIM), lambda i: (i, 0)),
        out_shape=jax.ShapeDtypeStruct((BATCH, EXPERT_DIM), jnp.float32),
    )(x, W, b2d)


@jax.jit
def kernel(expert_id, table, W, b):
    ids = expert_id.astype(jnp.int32).reshape(BATCH // CHUNK, CHUNK)
    gathered = _sc_gather_fn()(table, ids)
    return _tc_linear(gathered, W, b.reshape(1, EXPERT_DIM))
